# baseline (speedup 1.0000x reference)
import functools

import jax
import jax.numpy as jnp
from jax import lax
from jax.experimental import pallas as pl
from jax.experimental.pallas import tpu as pltpu
from jax.experimental.pallas import tpu_sc as plsc

N, D, H, C, E = 10000, 128, 128, 6, 320000
NC, NS = 2, 16
NW = NC * NS
K = 128
IDXB = 8
NCHUNK = 2 * IDXB * (-(-E // (NW * K * 2 * IDXB)))
NBANK = NCHUNK // IDXB
assert NBANK % 2 == 0 and NBANK >= 4
EPW = NCHUNK * K
E_PAD = NW * EPW
N_PAD = 10112
RPS = N_PAD // NS
ND_PAD = 10240
RPSD = ND_PAD // NS
PAD_DST_ROWS = N_PAD - N


def _make_sc_agg(width, with_deg):
    NRB = 2 if width == 128 else 8
    AHEAD = NRB // 2

    def body(*refs):
        if with_deg:
            (table, srci, dsti, out, deg_out,
             acc, deg_acc, src_ring, dst_ring, *rest) = refs
        else:
            (table, srci, dsti, out,
             acc, src_ring, dst_ring, *rest) = refs
        rows = rest[0:NRB]
        rest = rest[NRB:]
        if with_deg:
            ones_v, zcol_v = rest[0:2]
            rest = rest[2:]
        isems, idsems = rest[0:2], rest[2:4]
        gs, ss = rest[4:4 + NRB], rest[4 + NRB:4 + 2 * NRB]
        es = rest[4 + 2 * NRB:4 + 2 * NRB + 2] if with_deg else None
        rows_v = rows[0]
        c = lax.axis_index("c")
        s = lax.axis_index("s")
        wid = c * NS + s

        def zrow(i, _):
            for jj in range(width // 16):
                rows_v[i, pl.ds(jj * 16, 16)] = jnp.zeros((16,), jnp.float32)
            return _
        lax.fori_loop(0, K, zrow, None)
        zoffs = list(range(0, RPS - K + 1, K)) + ([RPS - K] if RPS % K else [])
        for off in zoffs:
            pltpu.sync_copy(rows_v, acc.at[pl.ds(s * RPS + off, K)])
        if with_deg:
            def zdeg(i, _):
                zcol_v[pl.ds(i * 16, 16)] = jnp.zeros((16,), jnp.float32)
                return _
            lax.fori_loop(0, RPSD // 16, zdeg, None)
            pltpu.sync_copy(zcol_v, deg_acc.at[pl.ds(s * RPSD, RPSD)])
            def ones(i, _):
                ones_v[pl.ds(i * 16, 16)] = jnp.ones((16,), jnp.float32)
                return _
            lax.fori_loop(0, K // 16, ones, None)

        plsc.subcore_barrier()

        def idx_load(bank, half):
            j0 = bank * IDXB
            pltpu.async_copy(srci.at[wid, pl.ds(j0, IDXB)],
                             src_ring.at[pl.ds(half * IDXB, IDXB)], isems[half])
            pltpu.async_copy(dsti.at[wid, pl.ds(j0, IDXB)],
                             dst_ring.at[pl.ds(half * IDXB, IDXB)], idsems[half])

        def wait_isrc(bank, half):
            j0 = bank * IDXB
            pltpu.make_async_copy(srci.at[wid, pl.ds(j0, IDXB)],
                                  src_ring.at[pl.ds(half * IDXB, IDXB)],
                                  isems[half]).wait()

        def wait_idst(bank, half):
            j0 = bank * IDXB
            pltpu.make_async_copy(dsti.at[wid, pl.ds(j0, IDXB)],
                                  dst_ring.at[pl.ds(half * IDXB, IDXB)],
                                  idsems[half]).wait()

        def gather(slot, b):
            pltpu.async_copy(table.at[src_ring.at[slot]], rows[b], gs[b])

        def wait_gather(slot, b):
            pltpu.make_async_copy(table.at[src_ring.at[slot]], rows[b],
                                  gs[b]).wait()

        def wait_scatter_of(slot, b):
            pltpu.make_async_copy(rows[b], acc.at[dst_ring.at[slot]],
                                  ss[b]).wait()
            if with_deg:
                pltpu.make_async_copy(ones_v, deg_acc.at[dst_ring.at[slot]],
                                      es[b]).wait()

        def bank_body(bank, half, load_next, last_bank, first_bank=False):
            wait_idst(bank, half)
            for dj in range(IDXB):
                j = bank * IDXB + dj
                if not (first_bank and dj == 0):
                    if dj > 0:
                        wait_scatter_of(half * IDXB + dj - 1, (dj - 1) % NRB)
                    else:
                        wait_scatter_of((1 - half) * IDXB + IDXB - 1,
                                        (IDXB - 1) % NRB)
                        if load_next:
                            idx_load(bank + 1, 1 - half)
                ja = dj + AHEAD
                if not (last_bank and ja >= IDXB):
                    if ja == IDXB:
                        wait_isrc(bank + 1, 1 - half)
                    gather((half if ja < IDXB else 1 - half) * IDXB + ja % IDXB,
                           (dj + AHEAD) % NRB)
                wait_gather(half * IDXB + dj, dj % NRB)
                pltpu.async_copy(rows[dj % NRB],
                                 acc.at[dst_ring.at[half * IDXB + dj]],
                                 ss[dj % NRB], add=True)
                if with_deg:
                    pltpu.async_copy(
                        ones_v, deg_acc.at[dst_ring.at[half * IDXB + dj]],
                        es[dj % NRB], add=True)

        idx_load(0, 0)
        idx_load(1, 1)
        wait_isrc(0, 0)
        for a in range(AHEAD):
            gather(a, a % NRB)

        bank_body(0, 0, False, False, first_bank=True)
        bank_body(1, 1, True, False)

        def steady(i, _):
            bank = 2 * i + 2
            bank_body(bank, 0, True, False)
            bank_body(bank + 1, 1, True, False)
            return _
        lax.fori_loop(0, NBANK // 2 - 2, steady, None)
        bank_body(NBANK - 2, 0, True, False)
        bank_body(NBANK - 1, 1, False, True)
        wait_scatter_of(IDXB + IDXB - 1, (IDXB - 1) % NRB)
        plsc.subcore_barrier()

        pltpu.sync_copy(acc.at[pl.ds(s * RPS, RPS)],
                        out.at[c, pl.ds(s * RPS, RPS)])
        if with_deg:
            pltpu.sync_copy(deg_acc.at[pl.ds(s * RPSD, RPSD)],
                            deg_out.at[c, pl.ds(s * RPSD, RPSD)])

    out_type = [jax.ShapeDtypeStruct((NC, N_PAD, width), jnp.float32)]
    scratch = [
        pltpu.VMEM_SHARED((N_PAD, width), jnp.float32),
    ]
    if with_deg:
        out_type.append(jax.ShapeDtypeStruct((NC, ND_PAD), jnp.float32))
        scratch.append(pltpu.VMEM_SHARED((ND_PAD,), jnp.float32))
    scratch += [
        pltpu.VMEM((2 * IDXB, K), jnp.int32),
        pltpu.VMEM((2 * IDXB, K), jnp.int32),
    ]
    scratch += [pltpu.VMEM((K, width), jnp.float32)] * NRB
    if with_deg:
        scratch += [
            pltpu.VMEM((K,), jnp.float32),
            pltpu.VMEM((RPSD,), jnp.float32),
        ]
    scratch += [pltpu.SemaphoreType.DMA] * (4 + 2 * NRB + (2 if with_deg else 0))
    return pl.kernel(
        body,
        out_type=tuple(out_type) if with_deg else out_type[0],
        mesh=plsc.VectorSubcoreMesh(core_axis_name="c", subcore_axis_name="s"),
        scratch_types=scratch,
        compiler_params=pltpu.CompilerParams(use_tc_tiling_on_sc=(width == 128)),
    )


_sc_agg1 = _make_sc_agg(D, with_deg=True)
_sc_agg2 = _make_sc_agg(H, with_deg=False)
_sc_agg3 = _make_sc_agg(16, with_deg=False)


def _dinv(degP):
    deg = degP[0] + degP[1]
    return 1.0 / jnp.maximum(deg, 1.0)


def _mm_t(a, w):
    return lax.dot_general(a, w, (((1,), (1,)), ((), ())),
                           preferred_element_type=jnp.float32)


BR = 1000


def _tc_layer1(P_ref, degP_ref, x_ref, Wl_ref, b_ref, Wr_ref, out_ref):
    agg = (P_ref[0] + P_ref[1]) * _dinv(degP_ref[...])
    out_ref[...] = jnp.maximum(
        _mm_t(agg, Wl_ref[...]) + b_ref[...][None, :] + _mm_t(x_ref[...], Wr_ref[...]),
        0.0)


def _tc_layer2(P_ref, degP_ref, x_ref, Wl_ref, b_ref, Wr_ref, W3lp_ref,
               out_ref, t_ref):
    agg = (P_ref[0] + P_ref[1]) * _dinv(degP_ref[...])
    h = jnp.maximum(
        _mm_t(agg, Wl_ref[...]) + b_ref[...][None, :] + _mm_t(x_ref[...], Wr_ref[...]),
        0.0)
    out_ref[...] = h
    t_ref[...] = _mm_t(h, W3lp_ref[...])


def _tc_layer3(R_ref, degP_ref, h_ref, b3_ref, W3r_ref, out_ref):
    agg = (R_ref[0, :, :C] + R_ref[1, :, :C]) * _dinv(degP_ref[...])
    out_ref[...] = agg + b3_ref[...][None, :] + _mm_t(h_ref[...], W3r_ref[...])


def _rows_spec(w):
    return pl.BlockSpec((BR, w), lambda i: (i, 0))


def _part_spec(w):
    return pl.BlockSpec((NC, BR, w), lambda i: (0, i, 0))


def _full_spec(shape):
    nd = len(shape)
    return pl.BlockSpec(shape, lambda i: (0,) * nd)


def kernel(x, edge_index, W1l, b1, W1r, W2l, b2, W2r, W3l, b3, W3r):
    src = edge_index[0].astype(jnp.int32)
    dst = edge_index[1].astype(jnp.int32)
    pad = E_PAD - E
    pad_ar = jnp.arange(pad, dtype=jnp.int32)
    src_p = jnp.concatenate([src, pad_ar % N])
    dst_p = jnp.concatenate([dst, N + pad_ar % PAD_DST_ROWS])
    src3 = src_p.reshape(NW, NCHUNK, K)
    dst3 = dst_p.reshape(NW, NCHUNK, K)
    W3lp = jnp.zeros((16, H), jnp.float32).at[:C, :].set(W3l)

    P, degP = _sc_agg1(x, src3, dst3)
    degP = degP.reshape(NC, ND_PAD, 1)
    h1 = pl.pallas_call(
        _tc_layer1,
        grid=(N // BR,),
        in_specs=[_part_spec(H), _part_spec(1), _rows_spec(D),
                  _full_spec((H, D)), _full_spec((H,)), _full_spec((H, D))],
        out_specs=_rows_spec(H),
        out_shape=jax.ShapeDtypeStruct((N, H), jnp.float32),
    )(P, degP, x, W1l, b1, W1r)

    Q = _sc_agg2(h1, src3, dst3)
    h2, t = pl.pallas_call(
        _tc_layer2,
        grid=(N // BR,),
        in_specs=[_part_spec(H), _part_spec(1), _rows_spec(H),
                  _full_spec((H, H)), _full_spec((H,)), _full_spec((H, H)),
                  _full_spec((16, H))],
        out_specs=(_rows_spec(H), _rows_spec(16)),
        out_shape=(jax.ShapeDtypeStruct((N, H), jnp.float32),
                   jax.ShapeDtypeStruct((N, 16), jnp.float32)),
    )(Q, degP, h1, W2l, b2, W2r, W3lp)

    R = _sc_agg3(t, src3, dst3)
    out = pl.pallas_call(
        _tc_layer3,
        grid=(N // BR,),
        in_specs=[_part_spec(16), _part_spec(1), _rows_spec(H),
                  _full_spec((C,)), _full_spec((C, H))],
        out_specs=_rows_spec(C),
        out_shape=jax.ShapeDtypeStruct((N, C), jnp.float32),
    )(R, degP, h2, b3, W3r)
    return out


# speedup vs baseline: 1.0335x; 1.0335x over previous
import functools

import jax
import jax.numpy as jnp
from jax import lax
from jax.experimental import pallas as pl
from jax.experimental.pallas import tpu as pltpu
from jax.experimental.pallas import tpu_sc as plsc

N, D, H, C, E = 10000, 128, 128, 6, 320000
NC, NS = 2, 16
NW = NC * NS
K = 128
IDXB = 8
NCHUNK = 2 * IDXB * (-(-E // (NW * K * 2 * IDXB)))
NBANK = NCHUNK // IDXB
assert NBANK % 2 == 0 and NBANK >= 4
EPW = NCHUNK * K
E_PAD = NW * EPW
N_PAD = 10112
RPS = N_PAD // NS
ND_PAD = 10240
RPSD = ND_PAD // NS
PAD_DST_ROWS = N_PAD - N


def _make_sc_agg(width, with_deg):
    NRB = 2 if width == 128 else 8
    AHEAD = NRB // 2

    def body(*refs):
        if with_deg:
            (table, srci, dsti, out, deg_out,
             acc, deg_acc, src_ring, dst_ring, *rest) = refs
        else:
            (table, srci, dsti, out,
             acc, src_ring, dst_ring, *rest) = refs
        rows = rest[0:NRB]
        rest = rest[NRB:]
        if with_deg:
            ones_v, zcol_v = rest[0:2]
            rest = rest[2:]
        isems, idsems = rest[0:2], rest[2:4]
        gs, ss = rest[4:4 + NRB], rest[4 + NRB:4 + 2 * NRB]
        es = rest[4 + 2 * NRB:4 + 2 * NRB + 2] if with_deg else None
        rows_v = rows[0]
        c = lax.axis_index("c")
        s = lax.axis_index("s")
        wid = c * NS + s

        def zrow(i, _):
            for jj in range(width // 16):
                rows_v[i, pl.ds(jj * 16, 16)] = jnp.zeros((16,), jnp.float32)
            return _
        lax.fori_loop(0, K, zrow, None)
        zoffs = list(range(0, RPS - K + 1, K)) + ([RPS - K] if RPS % K else [])
        for off in zoffs:
            pltpu.sync_copy(rows_v, acc.at[pl.ds(s * RPS + off, K)])
        if with_deg:
            def zdeg(i, _):
                zcol_v[pl.ds(i * 16, 16)] = jnp.zeros((16,), jnp.float32)
                return _
            lax.fori_loop(0, RPSD // 16, zdeg, None)
            pltpu.sync_copy(zcol_v, deg_acc.at[pl.ds(s * RPSD, RPSD)])
            def ones(i, _):
                ones_v[pl.ds(i * 16, 16)] = jnp.ones((16,), jnp.float32)
                return _
            lax.fori_loop(0, K // 16, ones, None)

        plsc.subcore_barrier()

        def idx_load(bank, half):
            j0 = bank * IDXB
            pltpu.async_copy(srci.at[wid, pl.ds(j0, IDXB)],
                             src_ring.at[pl.ds(half * IDXB, IDXB)], isems[half])
            pltpu.async_copy(dsti.at[wid, pl.ds(j0, IDXB)],
                             dst_ring.at[pl.ds(half * IDXB, IDXB)], idsems[half])

        def wait_isrc(bank, half):
            j0 = bank * IDXB
            pltpu.make_async_copy(srci.at[wid, pl.ds(j0, IDXB)],
                                  src_ring.at[pl.ds(half * IDXB, IDXB)],
                                  isems[half]).wait()

        def wait_idst(bank, half):
            j0 = bank * IDXB
            pltpu.make_async_copy(dsti.at[wid, pl.ds(j0, IDXB)],
                                  dst_ring.at[pl.ds(half * IDXB, IDXB)],
                                  idsems[half]).wait()

        def gather(slot, b):
            pltpu.async_copy(table.at[src_ring.at[slot]], rows[b], gs[b])

        def wait_gather(slot, b):
            pltpu.make_async_copy(table.at[src_ring.at[slot]], rows[b],
                                  gs[b]).wait()

        def wait_scatter_of(slot, b):
            pltpu.make_async_copy(rows[b], acc.at[dst_ring.at[slot]],
                                  ss[b]).wait()
            if with_deg:
                pltpu.make_async_copy(ones_v, deg_acc.at[dst_ring.at[slot]],
                                      es[b]).wait()

        def bank_body(bank, half, load_next, last_bank, first_bank=False):
            wait_idst(bank, half)
            for dj in range(IDXB):
                j = bank * IDXB + dj
                if not (first_bank and dj == 0):
                    if dj > 0:
                        wait_scatter_of(half * IDXB + dj - 1, (dj - 1) % NRB)
                    else:
                        wait_scatter_of((1 - half) * IDXB + IDXB - 1,
                                        (IDXB - 1) % NRB)
                        if load_next:
                            idx_load(bank + 1, 1 - half)
                ja = dj + AHEAD
                if not (last_bank and ja >= IDXB):
                    if ja == IDXB:
                        wait_isrc(bank + 1, 1 - half)
                    gather((half if ja < IDXB else 1 - half) * IDXB + ja % IDXB,
                           (dj + AHEAD) % NRB)
                wait_gather(half * IDXB + dj, dj % NRB)
                pltpu.async_copy(rows[dj % NRB],
                                 acc.at[dst_ring.at[half * IDXB + dj]],
                                 ss[dj % NRB], add=True)
                if with_deg:
                    pltpu.async_copy(
                        ones_v, deg_acc.at[dst_ring.at[half * IDXB + dj]],
                        es[dj % NRB], add=True)

        idx_load(0, 0)
        idx_load(1, 1)
        wait_isrc(0, 0)
        for a in range(AHEAD):
            gather(a, a % NRB)

        bank_body(0, 0, False, False, first_bank=True)
        bank_body(1, 1, True, False)

        def steady(i, _):
            bank = 2 * i + 2
            bank_body(bank, 0, True, False)
            bank_body(bank + 1, 1, True, False)
            return _
        lax.fori_loop(0, NBANK // 2 - 2, steady, None)
        bank_body(NBANK - 2, 0, True, False)
        bank_body(NBANK - 1, 1, False, True)
        wait_scatter_of(IDXB + IDXB - 1, (IDXB - 1) % NRB)
        plsc.subcore_barrier()

        pltpu.sync_copy(acc.at[pl.ds(s * RPS, RPS)],
                        out.at[c, pl.ds(s * RPS, RPS)])
        if with_deg:
            pltpu.sync_copy(deg_acc.at[pl.ds(s * RPSD, RPSD)],
                            deg_out.at[c, pl.ds(s * RPSD, RPSD)])

    out_type = [jax.ShapeDtypeStruct((NC, N_PAD, width), jnp.float32)]
    scratch = [
        pltpu.VMEM_SHARED((N_PAD, width), jnp.float32),
    ]
    if with_deg:
        out_type.append(jax.ShapeDtypeStruct((NC, ND_PAD), jnp.float32))
        scratch.append(pltpu.VMEM_SHARED((ND_PAD,), jnp.float32))
    scratch += [
        pltpu.VMEM((2 * IDXB, K), jnp.int32),
        pltpu.VMEM((2 * IDXB, K), jnp.int32),
    ]
    scratch += [pltpu.VMEM((K, width), jnp.float32)] * NRB
    if with_deg:
        scratch += [
            pltpu.VMEM((K,), jnp.float32),
            pltpu.VMEM((RPSD,), jnp.float32),
        ]
    scratch += [pltpu.SemaphoreType.DMA] * (4 + 2 * NRB + (2 if with_deg else 0))
    return pl.kernel(
        body,
        out_type=tuple(out_type) if with_deg else out_type[0],
        mesh=plsc.VectorSubcoreMesh(core_axis_name="c", subcore_axis_name="s"),
        scratch_types=scratch,
        compiler_params=pltpu.CompilerParams(use_tc_tiling_on_sc=(width == 128)),
    )


_sc_agg1 = _make_sc_agg(D, with_deg=True)
_sc_agg2 = _make_sc_agg(H, with_deg=False)
_sc_agg3 = _make_sc_agg(16, with_deg=False)


def _dinv(degP):
    deg = degP[:, :1] + degP[:, 1:2]
    return 1.0 / jnp.maximum(deg, 1.0)


def _mm_t(a, w):
    return lax.dot_general(a, w, (((1,), (1,)), ((), ())),
                           preferred_element_type=jnp.float32)


BR = 2000


def _tc_layer1(P_ref, degP_ref, x_ref, Wl_ref, b_ref, Wr_ref, out_ref):
    agg = (P_ref[0] + P_ref[1]) * _dinv(degP_ref[...])
    out_ref[...] = jnp.maximum(
        _mm_t(agg, Wl_ref[...]) + b_ref[...][None, :] + _mm_t(x_ref[...], Wr_ref[...]),
        0.0)


def _tc_layer2(P_ref, degP_ref, x_ref, Wl_ref, b_ref, Wr_ref, W3lp_ref,
               out_ref, t_ref):
    agg = (P_ref[0] + P_ref[1]) * _dinv(degP_ref[...])
    h = jnp.maximum(
        _mm_t(agg, Wl_ref[...]) + b_ref[...][None, :] + _mm_t(x_ref[...], Wr_ref[...]),
        0.0)
    out_ref[...] = h
    t_ref[...] = _mm_t(h, W3lp_ref[...])


def _tc_layer3(R_ref, degP_ref, h_ref, b3_ref, W3r_ref, out_ref):
    agg = (R_ref[0, :, :C] + R_ref[1, :, :C]) * _dinv(degP_ref[...])
    out_ref[...] = agg + b3_ref[...][None, :] + _mm_t(h_ref[...], W3r_ref[...])


def _rows_spec(w):
    return pl.BlockSpec((BR, w), lambda i: (i, 0))


def _part_spec(w):
    return pl.BlockSpec((NC, BR, w), lambda i: (0, i, 0))


def _deg_spec():
    return pl.BlockSpec((BR, NC), lambda i: (i, 0))


def _full_spec(shape):
    nd = len(shape)
    return pl.BlockSpec(shape, lambda i: (0,) * nd)


def kernel(x, edge_index, W1l, b1, W1r, W2l, b2, W2r, W3l, b3, W3r):
    src = edge_index[0].astype(jnp.int32)
    dst = edge_index[1].astype(jnp.int32)
    pad = E_PAD - E
    pad_ar = jnp.arange(pad, dtype=jnp.int32)
    src_p = jnp.concatenate([src, pad_ar % N])
    dst_p = jnp.concatenate([dst, N + pad_ar % PAD_DST_ROWS])
    src3 = src_p.reshape(NW, NCHUNK, K)
    dst3 = dst_p.reshape(NW, NCHUNK, K)
    W3lp = jnp.zeros((16, H), jnp.float32).at[:C, :].set(W3l)

    P, degP = _sc_agg1(x, src3, dst3)
    degP = degP.T
    h1 = pl.pallas_call(
        _tc_layer1,
        grid=(N // BR,),
        in_specs=[_part_spec(H), _deg_spec(), _rows_spec(D),
                  _full_spec((H, D)), _full_spec((H,)), _full_spec((H, D))],
        out_specs=_rows_spec(H),
        out_shape=jax.ShapeDtypeStruct((N, H), jnp.float32),
    )(P, degP, x, W1l, b1, W1r)

    Q = _sc_agg2(h1, src3, dst3)
    h2, t = pl.pallas_call(
        _tc_layer2,
        grid=(N // BR,),
        in_specs=[_part_spec(H), _deg_spec(), _rows_spec(H),
                  _full_spec((H, H)), _full_spec((H,)), _full_spec((H, H)),
                  _full_spec((16, H))],
        out_specs=(_rows_spec(H), _rows_spec(16)),
        out_shape=(jax.ShapeDtypeStruct((N, H), jnp.float32),
                   jax.ShapeDtypeStruct((N, 16), jnp.float32)),
    )(Q, degP, h1, W2l, b2, W2r, W3lp)

    R = _sc_agg3(t, src3, dst3)
    out = pl.pallas_call(
        _tc_layer3,
        grid=(N // BR,),
        in_specs=[_part_spec(16), _deg_spec(), _rows_spec(H),
                  _full_spec((C,)), _full_spec((C, H))],
        out_specs=_rows_spec(C),
        out_shape=jax.ShapeDtypeStruct((N, C), jnp.float32),
    )(R, degP, h2, b3, W3r)
    return out


# speedup vs baseline: 1.0337x; 1.0002x over previous
import jax
import jax.numpy as jnp
from jax import lax
from jax.experimental import pallas as pl
from jax.experimental.pallas import tpu as pltpu
from jax.experimental.pallas import tpu_sc as plsc

N, D, H, C, E = 10000, 128, 128, 6, 320000
NC, NS = 2, 16
NW = NC * NS
K = 128
IDXB = 8
NCHUNK = 2 * IDXB * (-(-E // (NW * K * 2 * IDXB)))
NBANK = NCHUNK // IDXB
assert NBANK % 2 == 0 and NBANK >= 4
EPW = NCHUNK * K
E_PAD = NW * EPW
N_PAD = 10112
RPS = N_PAD // NS
ND_PAD = 10240
RPSD = ND_PAD // NS
PAD_DST_ROWS = N_PAD - N


def _make_sc_agg(width, with_deg):
    NRB = 2 if width == 128 else 8
    AHEAD = NRB // 2

    def body(*refs):
        if with_deg:
            (table, srci, dsti, out, deg_out,
             acc, deg_acc, src_ring, dst_ring, *rest) = refs
        else:
            (table, srci, dsti, out,
             acc, src_ring, dst_ring, *rest) = refs
        rows = rest[0:NRB]
        rest = rest[NRB:]
        if with_deg:
            ones_v, zcol_v = rest[0:2]
            rest = rest[2:]
        isems, idsems = rest[0:2], rest[2:4]
        gs, ss = rest[4:4 + NRB], rest[4 + NRB:4 + 2 * NRB]
        es = rest[4 + 2 * NRB:4 + 2 * NRB + 2] if with_deg else None
        rows_v = rows[0]
        c = lax.axis_index("c")
        s = lax.axis_index("s")
        wid = c * NS + s

        def zrow(i, _):
            for jj in range(width // 16):
                rows_v[i, pl.ds(jj * 16, 16)] = jnp.zeros((16,), jnp.float32)
            return _
        lax.fori_loop(0, K, zrow, None)
        zoffs = list(range(0, RPS - K + 1, K)) + ([RPS - K] if RPS % K else [])
        for off in zoffs:
            pltpu.sync_copy(rows_v, acc.at[pl.ds(s * RPS + off, K)])
        if with_deg:
            def zdeg(i, _):
                zcol_v[pl.ds(i * 16, 16)] = jnp.zeros((16,), jnp.float32)
                return _
            lax.fori_loop(0, RPSD // 16, zdeg, None)
            pltpu.sync_copy(zcol_v, deg_acc.at[pl.ds(s * RPSD, RPSD)])
            def ones(i, _):
                ones_v[pl.ds(i * 16, 16)] = jnp.ones((16,), jnp.float32)
                return _
            lax.fori_loop(0, K // 16, ones, None)

        plsc.subcore_barrier()

        def idx_load(bank, half):
            j0 = bank * IDXB
            pltpu.async_copy(srci.at[wid, pl.ds(j0, IDXB)],
                             src_ring.at[pl.ds(half * IDXB, IDXB)], isems[half])
            pltpu.async_copy(dsti.at[wid, pl.ds(j0, IDXB)],
                             dst_ring.at[pl.ds(half * IDXB, IDXB)], idsems[half])

        def wait_isrc(bank, half):
            j0 = bank * IDXB
            pltpu.make_async_copy(srci.at[wid, pl.ds(j0, IDXB)],
                                  src_ring.at[pl.ds(half * IDXB, IDXB)],
                                  isems[half]).wait()

        def wait_idst(bank, half):
            j0 = bank * IDXB
            pltpu.make_async_copy(dsti.at[wid, pl.ds(j0, IDXB)],
                                  dst_ring.at[pl.ds(half * IDXB, IDXB)],
                                  idsems[half]).wait()

        def gather(slot, b):
            pltpu.async_copy(table.at[src_ring.at[slot]], rows[b], gs[b])

        def wait_gather(slot, b):
            pltpu.make_async_copy(table.at[src_ring.at[slot]], rows[b],
                                  gs[b]).wait()

        def wait_scatter_of(slot, b):
            pltpu.make_async_copy(rows[b], acc.at[dst_ring.at[slot]],
                                  ss[b]).wait()
            if with_deg:
                pltpu.make_async_copy(ones_v, deg_acc.at[dst_ring.at[slot]],
                                      es[b]).wait()

        def bank_body(bank, half, load_next, last_bank, first_bank=False):
            wait_idst(bank, half)
            for dj in range(IDXB):
                j = bank * IDXB + dj
                if not (first_bank and dj == 0):
                    if dj > 0:
                        wait_scatter_of(half * IDXB + dj - 1, (dj - 1) % NRB)
                    else:
                        wait_scatter_of((1 - half) * IDXB + IDXB - 1,
                                        (IDXB - 1) % NRB)
                        if load_next:
                            idx_load(bank + 1, 1 - half)
                ja = dj + AHEAD
                if not (last_bank and ja >= IDXB):
                    if ja == IDXB:
                        wait_isrc(bank + 1, 1 - half)
                    gather((half if ja < IDXB else 1 - half) * IDXB + ja % IDXB,
                           (dj + AHEAD) % NRB)
                wait_gather(half * IDXB + dj, dj % NRB)
                pltpu.async_copy(rows[dj % NRB],
                                 acc.at[dst_ring.at[half * IDXB + dj]],
                                 ss[dj % NRB], add=True)
                if with_deg:
                    pltpu.async_copy(
                        ones_v, deg_acc.at[dst_ring.at[half * IDXB + dj]],
                        es[dj % NRB], add=True)

        idx_load(0, 0)
        idx_load(1, 1)
        wait_isrc(0, 0)
        for a in range(AHEAD):
            gather(a, a % NRB)

        bank_body(0, 0, False, False, first_bank=True)
        bank_body(1, 1, True, False)

        def steady(i, _):
            bank = 2 * i + 2
            bank_body(bank, 0, True, False)
            bank_body(bank + 1, 1, True, False)
            return _
        lax.fori_loop(0, NBANK // 2 - 2, steady, None)
        bank_body(NBANK - 2, 0, True, False)
        bank_body(NBANK - 1, 1, False, True)
        wait_scatter_of(IDXB + IDXB - 1, (IDXB - 1) % NRB)
        plsc.subcore_barrier()

        pltpu.sync_copy(acc.at[pl.ds(s * RPS, RPS)],
                        out.at[c, pl.ds(s * RPS, RPS)])
        if with_deg:
            pltpu.sync_copy(deg_acc.at[pl.ds(s * RPSD, RPSD)],
                            deg_out.at[c, pl.ds(s * RPSD, RPSD)])

    out_type = [jax.ShapeDtypeStruct((NC, N_PAD, width), jnp.float32)]
    scratch = [
        pltpu.VMEM_SHARED((N_PAD, width), jnp.float32),
    ]
    if with_deg:
        out_type.append(jax.ShapeDtypeStruct((NC, ND_PAD), jnp.float32))
        scratch.append(pltpu.VMEM_SHARED((ND_PAD,), jnp.float32))
    scratch += [
        pltpu.VMEM((2 * IDXB, K), jnp.int32),
        pltpu.VMEM((2 * IDXB, K), jnp.int32),
    ]
    scratch += [pltpu.VMEM((K, width), jnp.float32)] * NRB
    if with_deg:
        scratch += [
            pltpu.VMEM((K,), jnp.float32),
            pltpu.VMEM((RPSD,), jnp.float32),
        ]
    scratch += [pltpu.SemaphoreType.DMA] * (4 + 2 * NRB + (2 if with_deg else 0))
    return pl.kernel(
        body,
        out_type=tuple(out_type) if with_deg else out_type[0],
        mesh=plsc.VectorSubcoreMesh(core_axis_name="c", subcore_axis_name="s"),
        scratch_types=scratch,
        compiler_params=pltpu.CompilerParams(use_tc_tiling_on_sc=(width == 128)),
    )


_sc_agg1 = _make_sc_agg(D, with_deg=True)
_sc_agg2 = _make_sc_agg(H, with_deg=False)
_sc_agg3 = _make_sc_agg(16, with_deg=False)


def _dinv(degP):
    deg = degP[:, :1] + degP[:, 1:2]
    return 1.0 / jnp.maximum(deg, 1.0)


def _mm_t(a, w):
    return lax.dot_general(a, w, (((1,), (1,)), ((), ())),
                           preferred_element_type=jnp.float32)


BR = 2000


def _tc_layer1(P_ref, degP_ref, x_ref, Wl_ref, b_ref, Wr_ref, out_ref):
    agg = (P_ref[0] + P_ref[1]) * _dinv(degP_ref[...])
    out_ref[...] = jnp.maximum(
        _mm_t(agg, Wl_ref[...]) + b_ref[...][None, :] + _mm_t(x_ref[...], Wr_ref[...]),
        0.0)


def _tc_layer2(P_ref, degP_ref, x_ref, Wl_ref, b_ref, Wr_ref, W3lp_ref,
               out_ref, t_ref):
    agg = (P_ref[0] + P_ref[1]) * _dinv(degP_ref[...])
    h = jnp.maximum(
        _mm_t(agg, Wl_ref[...]) + b_ref[...][None, :] + _mm_t(x_ref[...], Wr_ref[...]),
        0.0)
    out_ref[...] = h
    t_ref[...] = _mm_t(h, W3lp_ref[...])


def _tc_layer3(R_ref, degP_ref, h_ref, b3_ref, W3r_ref, out_ref):
    agg = (R_ref[0, :, :C] + R_ref[1, :, :C]) * _dinv(degP_ref[...])
    out_ref[...] = agg + b3_ref[...][None, :] + _mm_t(h_ref[...], W3r_ref[...])


def _rows_spec(w):
    return pl.BlockSpec((BR, w), lambda i: (i, 0))


def _part_spec(w):
    return pl.BlockSpec((NC, BR, w), lambda i: (0, i, 0))


def _deg_spec():
    return pl.BlockSpec((BR, NC), lambda i: (i, 0))


def _full_spec(shape):
    nd = len(shape)
    return pl.BlockSpec(shape, lambda i: (0,) * nd)


def kernel(x, edge_index, W1l, b1, W1r, W2l, b2, W2r, W3l, b3, W3r):
    src = edge_index[0].astype(jnp.int32)
    dst = edge_index[1].astype(jnp.int32)
    pad = E_PAD - E
    pad_ar = jnp.arange(pad, dtype=jnp.int32)
    src_p = jnp.concatenate([src, pad_ar % N])
    dst_p = jnp.concatenate([dst, N + pad_ar % PAD_DST_ROWS])
    src3 = src_p.reshape(NW, NCHUNK, K)
    dst3 = dst_p.reshape(NW, NCHUNK, K)
    W3lp = jnp.zeros((16, H), jnp.float32).at[:C, :].set(W3l)

    P, degP = _sc_agg1(x, src3, dst3)
    degP = degP.T
    h1 = pl.pallas_call(
        _tc_layer1,
        grid=(N // BR,),
        in_specs=[_part_spec(H), _deg_spec(), _rows_spec(D),
                  _full_spec((H, D)), _full_spec((H,)), _full_spec((H, D))],
        out_specs=_rows_spec(H),
        out_shape=jax.ShapeDtypeStruct((N, H), jnp.float32),
    )(P, degP, x, W1l, b1, W1r)

    Q = _sc_agg2(h1, src3, dst3)
    h2, t = pl.pallas_call(
        _tc_layer2,
        grid=(N // BR,),
        in_specs=[_part_spec(H), _deg_spec(), _rows_spec(H),
                  _full_spec((H, H)), _full_spec((H,)), _full_spec((H, H)),
                  _full_spec((16, H))],
        out_specs=(_rows_spec(H), _rows_spec(16)),
        out_shape=(jax.ShapeDtypeStruct((N, H), jnp.float32),
                   jax.ShapeDtypeStruct((N, 16), jnp.float32)),
    )(Q, degP, h1, W2l, b2, W2r, W3lp)

    R = _sc_agg3(t, src3, dst3)
    out = pl.pallas_call(
        _tc_layer3,
        grid=(N // BR,),
        in_specs=[_part_spec(16), _deg_spec(), _rows_spec(H),
                  _full_spec((C,)), _full_spec((C, H))],
        out_specs=_rows_spec(C),
        out_shape=jax.ShapeDtypeStruct((N, C), jnp.float32),
    )(R, degP, h2, b3, W3r)
    return out
